# baseline (speedup 1.0000x reference)
import jax
import jax.numpy as jnp
from jax import lax
from jax.experimental import pallas as pl
from jax.experimental.pallas import tpu as pltpu
from jax.experimental.pallas import tpu_sc as plsc

B = 16384
D = 64
K = 20
NCTX = K + 1
NC = 2
NS = 16
NW = NC * NS
BPW = B // NW
C = 32
NCHUNK = BPW // C
L = 16
IDX_DMA = 128
SSTR = 17
SCTX = L * SSTR


def _sc_scores_kernel(center_hbm, pos_hbm, neg_hbm, win_hbm, wout_hbm,
                      pos_out, neg_out,
                      cidx, pidx, nidx,
                      cbuf0, pbuf0, nbuf0, cbuf1, pbuf1, nbuf1,
                      scr, sbuf, sem0, sem1):
    c = lax.axis_index("c")
    s = lax.axis_index("s")
    wid = s * NC + c
    base = wid * BPW

    pltpu.sync_copy(center_hbm.at[pl.ds(base, BPW)], cidx)
    pltpu.sync_copy(pos_hbm.at[pl.ds(base, BPW)], pidx)
    pltpu.sync_copy(neg_hbm.at[pl.ds(base * K, BPW * K)], nidx)

    slots = ((cbuf0, pbuf0, nbuf0, sem0), (cbuf1, pbuf1, nbuf1, sem1))

    def copies(t, slot):
        cb, pb, nb, sem = slots[slot]
        cps = [
            (win_hbm.at[cidx.at[pl.ds(t * C, C)]], cb, sem),
            (wout_hbm.at[pidx.at[pl.ds(t * C, C)]], pb, sem),
        ]
        for j in range(C * K // IDX_DMA):
            cps.append((
                wout_hbm.at[nidx.at[pl.ds(t * C * K + j * IDX_DMA, IDX_DMA)]],
                nb.at[pl.ds(j * IDX_DMA, IDX_DMA)], sem))
        return cps

    def start(t, slot):
        for src, dst, sem in copies(t, slot):
            pltpu.async_copy(src, dst, sem)

    def drain(t, slot):
        for src, dst, sem in copies(t, slot):
            pltpu.make_async_copy(src, dst, sem).wait()

    iot = lax.iota(jnp.int32, L)

    def compute(t, slot):
        cb, pb, nb, _ = slots[slot]
        for g in range(C // L):
            grow = g * L

            def rbody(r, carry):
                row = grow + r
                sidx = iot * SSTR + r
                hs = [cb[row, pl.ds(j * L, L)] for j in range(D // L)]
                ps = [pb[row, pl.ds(j * L, L)] for j in range(D // L)]
                v = ((hs[0] * ps[0] + hs[1] * ps[1])
                     + (hs[2] * ps[2] + hs[3] * ps[3]))
                plsc.store_scatter(scr, [sidx], v)
                nrow = row * K
                for k in range(K):
                    es = [nb[nrow + k, pl.ds(j * L, L)]
                          for j in range(D // L)]
                    v = ((hs[0] * es[0] + hs[1] * es[1])
                         + (hs[2] * es[2] + hs[3] * es[3]))
                    plsc.store_scatter(scr, [sidx + (k + 1) * SCTX], v)
                return carry

            lax.fori_loop(0, L, rbody, 0)

            off = t * C + grow

            def cbody(ctx, carry):
                sb = ctx * SCTX
                acc = scr[pl.ds(sb, L)]
                for l in range(1, L):
                    acc = acc + scr[pl.ds(sb + l * SSTR, L)]
                sbuf[pl.ds(ctx * BPW + off, L)] = acc
                return carry

            lax.fori_loop(0, NCTX, cbody, 0)

    start(0, 0)
    start(1, 1)

    @pl.loop(0, NCHUNK, step=2)
    def _chunk_loop(t):
        for b in range(2):
            drain(t + b, b)
            compute(t + b, b)

            @pl.when(t + b + 2 < NCHUNK)
            def _prefetch():
                start(t + b + 2, b)

    pltpu.sync_copy(sbuf.at[pl.ds(0, BPW)], pos_out.at[pl.ds(base, BPW)])
    for k in range(K):
        pltpu.sync_copy(sbuf.at[pl.ds((k + 1) * BPW, BPW)],
                        neg_out.at[k, pl.ds(base, BPW)])


@jax.jit
def _sc_scores(center, pos_context, neg_flat, W_in, W_out):
    mesh = plsc.VectorSubcoreMesh(core_axis_name="c", subcore_axis_name="s",
                                  num_cores=NC, num_subcores=NS)
    return pl.kernel(
        _sc_scores_kernel,
        out_type=(jax.ShapeDtypeStruct((B,), jnp.float32),
                  jax.ShapeDtypeStruct((K, B), jnp.float32)),
        mesh=mesh,
        compiler_params=pltpu.CompilerParams(needs_layout_passes=False,
                                             use_tc_tiling_on_sc=False),
        scratch_types=[
            pltpu.VMEM((BPW,), jnp.int32),
            pltpu.VMEM((BPW,), jnp.int32),
            pltpu.VMEM((BPW * K,), jnp.int32),
            pltpu.VMEM((C, D), jnp.float32),
            pltpu.VMEM((C, D), jnp.float32),
            pltpu.VMEM((C * K, D), jnp.float32),
            pltpu.VMEM((C, D), jnp.float32),
            pltpu.VMEM((C, D), jnp.float32),
            pltpu.VMEM((C * K, D), jnp.float32),
            pltpu.VMEM((NCTX * SCTX,), jnp.float32),
            pltpu.VMEM((NCTX * BPW,), jnp.float32),
            pltpu.SemaphoreType.DMA,
            pltpu.SemaphoreType.DMA,
        ],
    )(center, pos_context, neg_flat, W_in, W_out)


def _loss_body(pos_ref, neg_ref, out_ref):
    p = pos_ref[...]
    n = neg_ref[...]
    total = jnp.sum(jax.nn.softplus(-p)) + jnp.sum(jax.nn.softplus(n))
    out_ref[0, 0] = total / jnp.float32(B)


@jax.jit
def _tc_loss(pos_score, neg_score):
    out = pl.pallas_call(
        _loss_body,
        out_shape=jax.ShapeDtypeStruct((1, 1), jnp.float32),
        out_specs=pl.BlockSpec(memory_space=pltpu.SMEM),
    )(pos_score.reshape(B // 128, 128), neg_score.reshape(K * B // 128, 128))
    return out[0, 0]


def kernel(center, pos_context, neg_context, W_in, W_out):
    center = center.astype(jnp.int32)
    pos_context = pos_context.astype(jnp.int32)
    neg_flat = neg_context.astype(jnp.int32).reshape(-1)
    pos_score, neg_score = _sc_scores(center, pos_context, neg_flat,
                                      W_in, W_out)
    return _tc_loss(pos_score, neg_score)
